# baseline (speedup 1.0000x reference)
import functools

import jax
import jax.numpy as jnp
from jax import lax
from jax.experimental import pallas as pl
from jax.experimental.pallas import tpu as pltpu
from jax.experimental.pallas import tpu_sc as plsc


_EPS = 1e-5
_L = 16


def _rsqrt_newton(v):
    i = plsc.bitcast(v, jnp.int32)
    i = jnp.int32(0x5F3759DF) - lax.shift_right_logical(i, 1)
    r = plsc.bitcast(i, jnp.float32)
    h = v * -0.5
    for _ in range(3):
        r = r * (r * r * h + 1.5)
    return r


def _sc_body(in_hbm, pos_hbm, g_hbm, b_hbm, out_hbm,
             in0, in1, pos0, pos1, ou0, ou1, xbuf, gbuf, bbuf, sbuf, cbuf,
             sem_i0, sem_i1, sem_o0, sem_o1, sem_p0, sem_p1):
    info = plsc.get_sparse_core_info()
    nc = info.num_cores
    wid = lax.axis_index("s") * nc + lax.axis_index("c")
    batch = in_hbm.shape[0]
    seq = in_hbm.shape[1]
    hidden = in_hbm.shape[2]
    nw = nc * info.num_subcores
    rows_per_w = seq // nw
    chunk = in0.shape[0]
    njc = hidden // _L
    nsub = rows_per_w // chunk
    seq0 = wid * rows_per_w

    ins = (in0, in1)
    poss = (pos0, pos1)
    ous = (ou0, ou1)
    sems_i = (sem_i0, sem_i1)
    sems_o = (sem_o0, sem_o1)
    sems_p = (sem_p0, sem_p1)

    pltpu.sync_copy(g_hbm, gbuf)
    pltpu.sync_copy(b_hbm, bbuf)

    def compute(inb, posb, outb):
        @plsc.parallel_loop(0, chunk, unroll=2)
        def stats_row(r):
            def acc(j, carry):
                vs, vq = carry
                x = inb[r, pl.ds(j * _L, _L)] + posb[r, pl.ds(j * _L, _L)]
                xbuf[r, pl.ds(j * _L, _L)] = x
                return (vs + x, vq + x * x)

            zero = jnp.zeros((_L,), jnp.float32)
            vs, vq = lax.fori_loop(0, njc, acc, (zero, zero), unroll=8)
            lanes = jax.lax.iota(jnp.int32, _L)
            dnums = lax.GatherDimensionNumbers(
                offset_dims=(), collapsed_slice_dims=(0,),
                start_index_map=(0,))
            for k in (8, 4, 2, 1):
                perm = (lanes ^ k)[:, None]
                vs = vs + lax.gather(
                    vs, perm, dnums, (1,),
                    mode=lax.GatherScatterMode.PROMISE_IN_BOUNDS)
                vq = vq + lax.gather(
                    vq, perm, dnums, (1,),
                    mode=lax.GatherScatterMode.PROMISE_IN_BOUNDS)
            msp = vs * (1.0 / hidden)
            qsp = vq * (1.0 / hidden)
            var = qsp - msp * msp + _EPS
            rstd = _rsqrt_newton(var)
            sbuf[r, :] = rstd
            cbuf[r, :] = -(msp * rstd)

        lanes = jax.lax.iota(jnp.int32, _L)
        zidx = jnp.zeros((_L,), jnp.int32)
        svec = plsc.load_gather(sbuf, [lanes, zidx])
        cvec = plsc.load_gather(cbuf, [lanes, zidx])
        dnums = lax.GatherDimensionNumbers(
            offset_dims=(), collapsed_slice_dims=(0,), start_index_map=(0,))

        def col_loop(j, _):
            g = gbuf[pl.ds(j * _L, _L)]
            bb = bbuf[pl.ds(j * _L, _L)]

            @plsc.parallel_loop(0, chunk, unroll=4)
            def row_loop(r):
                ridx = jnp.full((_L, 1), r, jnp.int32)
                sr = lax.gather(svec, ridx, dnums, (1,),
                                mode=lax.GatherScatterMode.PROMISE_IN_BOUNDS)
                cr = lax.gather(cvec, ridx, dnums, (1,),
                                mode=lax.GatherScatterMode.PROMISE_IN_BOUNDS)
                x = xbuf[r, pl.ds(j * _L, _L)]
                outb[r, pl.ds(j * _L, _L)] = x * (sr * g) + (cr * g + bb)

            return 0

        lax.fori_loop(0, njc, col_loop, 0)

    pltpu.async_copy(in_hbm.at[0, pl.ds(seq0, chunk), :], ins[0], sems_i[0])
    pltpu.async_copy(pos_hbm.at[pl.ds(seq0, chunk), :], poss[0], sems_p[0])

    def do_sub(sub, p, not_first, not_last):
        row0 = seq0 + sub * chunk
        pltpu.make_async_copy(
            pos_hbm.at[pl.ds(row0, chunk), :], poss[p], sems_p[p]).wait()

        @pl.when(sub + 1 < nsub)
        def _():
            pltpu.async_copy(
                pos_hbm.at[pl.ds(row0 + chunk, chunk), :],
                poss[1 - p], sems_p[1 - p])
        for b in range(4):
            q = b & 1
            if b < 3:
                pltpu.async_copy(
                    in_hbm.at[b + 1, pl.ds(row0, chunk), :],
                    ins[(b + 1) & 1], sems_i[(b + 1) & 1])
            else:
                @pl.when(sub + 1 < nsub)
                def _():
                    pltpu.async_copy(
                        in_hbm.at[0, pl.ds(row0 + chunk, chunk), :],
                        ins[0], sems_i[0])
            pltpu.make_async_copy(
                in_hbm.at[b, pl.ds(row0, chunk), :], ins[q], sems_i[q]).wait()
            if b >= 2 or not_first:
                pltpu.make_async_copy(
                    ous[q], out_hbm.at[b, pl.ds(row0, chunk), :],
                    sems_o[q]).wait()
            else:
                @pl.when(sub > 0)
                def _():
                    pltpu.make_async_copy(
                        ous[q], out_hbm.at[b, pl.ds(row0, chunk), :],
                        sems_o[q]).wait()
            compute(ins[q], poss[p], ous[q])
            pltpu.async_copy(
                ous[q], out_hbm.at[b, pl.ds(row0, chunk), :], sems_o[q])

    def pair_loop(t, _):
        do_sub(2 * t, 0, not_first=False, not_last=True)
        do_sub(2 * t + 1, 1, not_first=True, not_last=False)
        return 0

    lax.fori_loop(0, nsub // 2, pair_loop, 0)
    for q in range(2):
        pltpu.make_async_copy(
            ous[q], out_hbm.at[0, pl.ds(seq0, chunk), :], sems_o[q]).wait()


@jax.jit
def _sc_kernel(input_feat, pos_table, ln_gamma, ln_beta):
    batch, seq, hidden = input_feat.shape
    chunk = 16
    mesh = plsc.VectorSubcoreMesh(core_axis_name="c", subcore_axis_name="s")
    run = pl.kernel(
        _sc_body,
        mesh=mesh,
        compiler_params=pltpu.CompilerParams(needs_layout_passes=False),
        out_type=jax.ShapeDtypeStruct((batch, seq, hidden), jnp.float32),
        scratch_types=[
            pltpu.VMEM((chunk, hidden), jnp.float32),
            pltpu.VMEM((chunk, hidden), jnp.float32),
            pltpu.VMEM((chunk, hidden), jnp.float32),
            pltpu.VMEM((chunk, hidden), jnp.float32),
            pltpu.VMEM((chunk, hidden), jnp.float32),
            pltpu.VMEM((chunk, hidden), jnp.float32),
            pltpu.VMEM((chunk, hidden), jnp.float32),
            pltpu.VMEM((hidden,), jnp.float32),
            pltpu.VMEM((hidden,), jnp.float32),
            pltpu.VMEM((chunk, _L), jnp.float32),
            pltpu.VMEM((chunk, _L), jnp.float32),
            pltpu.SemaphoreType.DMA,
            pltpu.SemaphoreType.DMA,
            pltpu.SemaphoreType.DMA,
            pltpu.SemaphoreType.DMA,
            pltpu.SemaphoreType.DMA,
            pltpu.SemaphoreType.DMA,
        ],
    )
    return run(input_feat, pos_table, ln_gamma, ln_beta)


def kernel(input_feat, pos_table, ln_gamma, ln_beta):
    seq = input_feat.shape[1]
    return _sc_kernel(input_feat, pos_table[:seq], ln_gamma, ln_beta)


# speedup vs baseline: 1.7918x; 1.7918x over previous
import functools

import jax
import jax.numpy as jnp
from jax import lax
from jax.experimental import pallas as pl
from jax.experimental.pallas import tpu as pltpu
from jax.experimental.pallas import tpu_sc as plsc


_EPS = 1e-5
_L = 16


def _rsqrt_newton(v):
    i = plsc.bitcast(v, jnp.int32)
    i = jnp.int32(0x5F3759DF) - lax.shift_right_logical(i, 1)
    r = plsc.bitcast(i, jnp.float32)
    h = v * -0.5
    for _ in range(3):
        r = r * (r * r * h + 1.5)
    return r


def _sc_body(in_hbm, pos_hbm, g_hbm, b_hbm, out_hbm,
             in0, in1, pos0, pos1, ou0, ou1, gbuf, bbuf, sbuf, cbuf,
             sem_i0, sem_i1, sem_o0, sem_o1, sem_p0, sem_p1):
    info = plsc.get_sparse_core_info()
    nc = info.num_cores
    wid = lax.axis_index("s") * nc + lax.axis_index("c")
    batch = in_hbm.shape[0]
    seq = in_hbm.shape[1]
    hidden = in_hbm.shape[2]
    nw = nc * info.num_subcores
    rows_per_w = seq // nw
    chunk = in0.shape[0]
    njc = hidden // _L
    nsub = rows_per_w // chunk
    seq0 = wid * rows_per_w

    ins = (in0, in1)
    poss = (pos0, pos1)
    ous = (ou0, ou1)
    sems_i = (sem_i0, sem_i1)
    sems_o = (sem_o0, sem_o1)
    sems_p = (sem_p0, sem_p1)

    pltpu.sync_copy(g_hbm, gbuf)
    pltpu.sync_copy(b_hbm, bbuf)

    def compute(inb, posb, outb):
        @plsc.parallel_loop(0, chunk, unroll=2)
        def stats_row(r):
            def acc(j, carry):
                vs, vq = carry
                x = inb[r, pl.ds(j * _L, _L)] + posb[r, pl.ds(j * _L, _L)]
                return (vs + x, vq + x * x)

            zero = jnp.zeros((_L,), jnp.float32)
            vs, vq = lax.fori_loop(0, njc, acc, (zero, zero), unroll=8)
            sbuf[r, :] = vs
            cbuf[r, :] = vq

        lanes = jax.lax.iota(jnp.int32, _L)
        vs_tot = jnp.zeros((_L,), jnp.float32)
        vq_tot = jnp.zeros((_L,), jnp.float32)
        for c in range(_L):
            cidx = jnp.full((_L,), c, jnp.int32)
            vs_tot = vs_tot + plsc.load_gather(sbuf, [lanes, cidx])
            vq_tot = vq_tot + plsc.load_gather(cbuf, [lanes, cidx])
        msp = vs_tot * (1.0 / hidden)
        qsp = vq_tot * (1.0 / hidden)
        var = qsp - msp * msp + _EPS
        rstd = _rsqrt_newton(var)
        svec = rstd
        cvec = -(msp * rstd)

        dnums = lax.GatherDimensionNumbers(
            offset_dims=(), collapsed_slice_dims=(0,), start_index_map=(0,))

        def col_loop(j, _):
            g = gbuf[pl.ds(j * _L, _L)]
            bb = bbuf[pl.ds(j * _L, _L)]

            @plsc.parallel_loop(0, chunk, unroll=4)
            def row_loop(r):
                ridx = jnp.full((_L, 1), r, jnp.int32)
                sr = lax.gather(svec, ridx, dnums, (1,),
                                mode=lax.GatherScatterMode.PROMISE_IN_BOUNDS)
                cr = lax.gather(cvec, ridx, dnums, (1,),
                                mode=lax.GatherScatterMode.PROMISE_IN_BOUNDS)
                x = inb[r, pl.ds(j * _L, _L)] + posb[r, pl.ds(j * _L, _L)]
                outb[r, pl.ds(j * _L, _L)] = x * (sr * g) + (cr * g + bb)

            return 0

        lax.fori_loop(0, njc, col_loop, 0)

    pltpu.async_copy(in_hbm.at[0, pl.ds(seq0, chunk), :], ins[0], sems_i[0])
    pltpu.async_copy(pos_hbm.at[pl.ds(seq0, chunk), :], poss[0], sems_p[0])

    def do_sub(sub, p, not_first, not_last):
        row0 = seq0 + sub * chunk
        pltpu.make_async_copy(
            pos_hbm.at[pl.ds(row0, chunk), :], poss[p], sems_p[p]).wait()

        @pl.when(sub + 1 < nsub)
        def _():
            pltpu.async_copy(
                pos_hbm.at[pl.ds(row0 + chunk, chunk), :],
                poss[1 - p], sems_p[1 - p])
        for b in range(4):
            q = b & 1
            if b < 3:
                pltpu.async_copy(
                    in_hbm.at[b + 1, pl.ds(row0, chunk), :],
                    ins[(b + 1) & 1], sems_i[(b + 1) & 1])
            else:
                @pl.when(sub + 1 < nsub)
                def _():
                    pltpu.async_copy(
                        in_hbm.at[0, pl.ds(row0 + chunk, chunk), :],
                        ins[0], sems_i[0])
            pltpu.make_async_copy(
                in_hbm.at[b, pl.ds(row0, chunk), :], ins[q], sems_i[q]).wait()
            if b >= 2 or not_first:
                pltpu.make_async_copy(
                    ous[q], out_hbm.at[b, pl.ds(row0, chunk), :],
                    sems_o[q]).wait()
            else:
                @pl.when(sub > 0)
                def _():
                    pltpu.make_async_copy(
                        ous[q], out_hbm.at[b, pl.ds(row0, chunk), :],
                        sems_o[q]).wait()
            compute(ins[q], poss[p], ous[q])
            pltpu.async_copy(
                ous[q], out_hbm.at[b, pl.ds(row0, chunk), :], sems_o[q])

    def pair_loop(t, _):
        do_sub(2 * t, 0, not_first=False, not_last=True)
        do_sub(2 * t + 1, 1, not_first=True, not_last=False)
        return 0

    lax.fori_loop(0, nsub // 2, pair_loop, 0)
    for q in range(2):
        pltpu.make_async_copy(
            ous[q], out_hbm.at[0, pl.ds(seq0, chunk), :], sems_o[q]).wait()


@jax.jit
def _sc_kernel(input_feat, pos_table, ln_gamma, ln_beta):
    batch, seq, hidden = input_feat.shape
    chunk = 16
    mesh = plsc.VectorSubcoreMesh(core_axis_name="c", subcore_axis_name="s")
    run = pl.kernel(
        _sc_body,
        mesh=mesh,
        compiler_params=pltpu.CompilerParams(needs_layout_passes=False),
        out_type=jax.ShapeDtypeStruct((batch, seq, hidden), jnp.float32),
        scratch_types=[
            pltpu.VMEM((chunk, hidden), jnp.float32),
            pltpu.VMEM((chunk, hidden), jnp.float32),
            pltpu.VMEM((chunk, hidden), jnp.float32),
            pltpu.VMEM((chunk, hidden), jnp.float32),
            pltpu.VMEM((chunk, hidden), jnp.float32),
            pltpu.VMEM((chunk, hidden), jnp.float32),
            pltpu.VMEM((hidden,), jnp.float32),
            pltpu.VMEM((hidden,), jnp.float32),
            pltpu.VMEM((chunk, _L), jnp.float32),
            pltpu.VMEM((chunk, _L), jnp.float32),
            pltpu.SemaphoreType.DMA,
            pltpu.SemaphoreType.DMA,
            pltpu.SemaphoreType.DMA,
            pltpu.SemaphoreType.DMA,
            pltpu.SemaphoreType.DMA,
            pltpu.SemaphoreType.DMA,
        ],
    )
    return run(input_feat, pos_table, ln_gamma, ln_beta)


def kernel(input_feat, pos_table, ln_gamma, ln_beta):
    seq = input_feat.shape[1]
    return _sc_kernel(input_feat, pos_table[:seq], ln_gamma, ln_beta)


# speedup vs baseline: 1.7995x; 1.0043x over previous
import functools

import jax
import jax.numpy as jnp
from jax import lax
from jax.experimental import pallas as pl
from jax.experimental.pallas import tpu as pltpu
from jax.experimental.pallas import tpu_sc as plsc


_EPS = 1e-5
_L = 16


def _rsqrt_newton(v):
    i = plsc.bitcast(v, jnp.int32)
    i = jnp.int32(0x5F3759DF) - lax.shift_right_logical(i, 1)
    r = plsc.bitcast(i, jnp.float32)
    h = v * -0.5
    for _ in range(3):
        r = r * (r * r * h + 1.5)
    return r


def _sc_body(in_hbm, pos_hbm, g_hbm, b_hbm, out_hbm,
             in0, in1, pos0, pos1, ou0, ou1, gbuf, bbuf, sbuf, cbuf,
             sem_i0, sem_i1, sem_o0, sem_o1, sem_p0, sem_p1):
    info = plsc.get_sparse_core_info()
    nc = info.num_cores
    wid = lax.axis_index("s") * nc + lax.axis_index("c")
    batch = in_hbm.shape[0]
    seq = in_hbm.shape[1]
    hidden = in_hbm.shape[2]
    nw = nc * info.num_subcores
    rows_per_w = seq // nw
    chunk = in0.shape[0]
    njc = hidden // _L
    nsub = rows_per_w // chunk
    seq0 = wid * rows_per_w

    ins = (in0, in1)
    poss = (pos0, pos1)
    ous = (ou0, ou1)
    sems_i = (sem_i0, sem_i1)
    sems_o = (sem_o0, sem_o1)
    sems_p = (sem_p0, sem_p1)

    pltpu.sync_copy(g_hbm, gbuf)
    pltpu.sync_copy(b_hbm, bbuf)

    def compute(inb, posb, outb):
        @plsc.parallel_loop(0, chunk, unroll=2)
        def stats_row(r):
            def acc(j, carry):
                vs, vq = carry
                x = inb[r, pl.ds(j * _L, _L)] + posb[r, pl.ds(j * _L, _L)]
                return (vs + x, vq + x * x)

            zero = jnp.zeros((_L,), jnp.float32)
            vs, vq = lax.fori_loop(0, njc, acc, (zero, zero), unroll=8)
            sbuf[r, :] = vs
            cbuf[r, :] = vq

        lanes = jax.lax.iota(jnp.int32, _L)
        vs_tot = jnp.zeros((_L,), jnp.float32)
        vq_tot = jnp.zeros((_L,), jnp.float32)
        for c in range(_L):
            cidx = jnp.full((_L,), c, jnp.int32)
            vs_tot = vs_tot + plsc.load_gather(sbuf, [lanes, cidx])
            vq_tot = vq_tot + plsc.load_gather(cbuf, [lanes, cidx])
        msp = vs_tot * (1.0 / hidden)
        qsp = vq_tot * (1.0 / hidden)
        var = qsp - msp * msp + _EPS
        rstd = _rsqrt_newton(var)
        svec = rstd
        cvec = -(msp * rstd)

        dnums = lax.GatherDimensionNumbers(
            offset_dims=(), collapsed_slice_dims=(0,), start_index_map=(0,))

        def col_loop(j, _):
            g = gbuf[pl.ds(j * _L, _L)]
            bb = bbuf[pl.ds(j * _L, _L)]

            @plsc.parallel_loop(0, chunk, unroll=4)
            def row_loop(r):
                ridx = jnp.full((_L, 1), r, jnp.int32)
                sr = lax.gather(svec, ridx, dnums, (1,),
                                mode=lax.GatherScatterMode.PROMISE_IN_BOUNDS)
                cr = lax.gather(cvec, ridx, dnums, (1,),
                                mode=lax.GatherScatterMode.PROMISE_IN_BOUNDS)
                x = inb[r, pl.ds(j * _L, _L)] + posb[r, pl.ds(j * _L, _L)]
                outb[r, pl.ds(j * _L, _L)] = x * (sr * g) + (cr * g + bb)

            return 0

        lax.fori_loop(0, njc, col_loop, 0, unroll=2)

    pltpu.async_copy(in_hbm.at[0, pl.ds(seq0, chunk), :], ins[0], sems_i[0])
    pltpu.async_copy(pos_hbm.at[pl.ds(seq0, chunk), :], poss[0], sems_p[0])

    def do_sub(sub, p, not_first, not_last):
        row0 = seq0 + sub * chunk
        pltpu.make_async_copy(
            pos_hbm.at[pl.ds(row0, chunk), :], poss[p], sems_p[p]).wait()

        @pl.when(sub + 1 < nsub)
        def _():
            pltpu.async_copy(
                pos_hbm.at[pl.ds(row0 + chunk, chunk), :],
                poss[1 - p], sems_p[1 - p])
        for b in range(4):
            q = b & 1
            if b < 3:
                pltpu.async_copy(
                    in_hbm.at[b + 1, pl.ds(row0, chunk), :],
                    ins[(b + 1) & 1], sems_i[(b + 1) & 1])
            else:
                @pl.when(sub + 1 < nsub)
                def _():
                    pltpu.async_copy(
                        in_hbm.at[0, pl.ds(row0 + chunk, chunk), :],
                        ins[0], sems_i[0])
            pltpu.make_async_copy(
                in_hbm.at[b, pl.ds(row0, chunk), :], ins[q], sems_i[q]).wait()
            if b >= 2 or not_first:
                pltpu.make_async_copy(
                    ous[q], out_hbm.at[b, pl.ds(row0, chunk), :],
                    sems_o[q]).wait()
            else:
                @pl.when(sub > 0)
                def _():
                    pltpu.make_async_copy(
                        ous[q], out_hbm.at[b, pl.ds(row0, chunk), :],
                        sems_o[q]).wait()
            compute(ins[q], poss[p], ous[q])
            pltpu.async_copy(
                ous[q], out_hbm.at[b, pl.ds(row0, chunk), :], sems_o[q])

    def pair_loop(t, _):
        do_sub(2 * t, 0, not_first=False, not_last=True)
        do_sub(2 * t + 1, 1, not_first=True, not_last=False)
        return 0

    lax.fori_loop(0, nsub // 2, pair_loop, 0)
    for q in range(2):
        pltpu.make_async_copy(
            ous[q], out_hbm.at[0, pl.ds(seq0, chunk), :], sems_o[q]).wait()


@jax.jit
def _sc_kernel(input_feat, pos_table, ln_gamma, ln_beta):
    batch, seq, hidden = input_feat.shape
    chunk = 16
    mesh = plsc.VectorSubcoreMesh(core_axis_name="c", subcore_axis_name="s")
    run = pl.kernel(
        _sc_body,
        mesh=mesh,
        compiler_params=pltpu.CompilerParams(needs_layout_passes=False),
        out_type=jax.ShapeDtypeStruct((batch, seq, hidden), jnp.float32),
        scratch_types=[
            pltpu.VMEM((chunk, hidden), jnp.float32),
            pltpu.VMEM((chunk, hidden), jnp.float32),
            pltpu.VMEM((chunk, hidden), jnp.float32),
            pltpu.VMEM((chunk, hidden), jnp.float32),
            pltpu.VMEM((chunk, hidden), jnp.float32),
            pltpu.VMEM((chunk, hidden), jnp.float32),
            pltpu.VMEM((hidden,), jnp.float32),
            pltpu.VMEM((hidden,), jnp.float32),
            pltpu.VMEM((chunk, _L), jnp.float32),
            pltpu.VMEM((chunk, _L), jnp.float32),
            pltpu.SemaphoreType.DMA,
            pltpu.SemaphoreType.DMA,
            pltpu.SemaphoreType.DMA,
            pltpu.SemaphoreType.DMA,
            pltpu.SemaphoreType.DMA,
            pltpu.SemaphoreType.DMA,
        ],
    )
    return run(input_feat, pos_table, ln_gamma, ln_beta)


def kernel(input_feat, pos_table, ln_gamma, ln_beta):
    seq = input_feat.shape[1]
    return _sc_kernel(input_feat, pos_table[:seq], ln_gamma, ln_beta)
